# baseline (speedup 1.0000x reference)
import functools

import jax
import jax.numpy as jnp
from jax import lax
from jax.experimental import pallas as pl
from jax.experimental.pallas import tpu as pltpu
from jax.experimental.pallas import tpu_sc as plsc

EMBED = 64
SCALE = 8.0
NC, NS = 2, 16
G0, G1 = 128, 72


def _rsqrt_newton(x):
    i = plsc.bitcast(x, jnp.int32)
    i = jnp.int32(0x5F3759DF) - lax.shift_right_logical(i, 1)
    y = plsc.bitcast(i, jnp.float32)
    for _ in range(3):
        y = y * (jnp.float32(1.5) - jnp.float32(0.5) * x * y * y)
    return y


def _normalize(big_v, offs_v, out_v, rb, n):
    lanes = lax.iota(jnp.int32, 16)
    perm = [lanes ^ (1 << k) for k in range(4)]

    @plsc.parallel_loop(0, n, unroll=8)
    def _row(r):
        o = offs_v[pl.ds(r, 16)][0]
        vs = [big_v[r, pl.ds(o + k * 16, 16)] for k in range(EMBED // 16)]
        ssv = vs[0] * vs[0]
        for v in vs[1:]:
            ssv = ssv + v * v
        for p in perm:
            ssv = ssv + jnp.take(ssv, p)
        x = jnp.maximum(ssv, jnp.float32(1e-30))
        norm = x * _rsqrt_newton(x)
        factor = jnp.float32(SCALE) / jnp.maximum(norm, jnp.float32(1e-12))
        for k, v in enumerate(vs):
            out_v[rb + r, pl.ds(k * 16, 16)] = v * factor


def _sc_embed(idx_flat, w2, nb, t):
    NW = NC * NS
    bpw = nb // NW
    ipw = bpw * t

    mesh = plsc.VectorSubcoreMesh(core_axis_name="c", subcore_axis_name="s")

    @functools.partial(
        pl.kernel,
        out_type=jax.ShapeDtypeStruct((nb, t, EMBED), jnp.float32),
        mesh=mesh,
        scratch_types=[
            pltpu.VMEM((ipw,), jnp.int32),
            pltpu.VMEM((G0, 2 * EMBED), jnp.float32),
            pltpu.VMEM((G1, 2 * EMBED), jnp.float32),
            pltpu.VMEM((G0, 2 * EMBED), jnp.float32),
            pltpu.VMEM((G1, 2 * EMBED), jnp.float32),
            pltpu.VMEM((t, EMBED), jnp.float32),
            pltpu.VMEM((t, EMBED), jnp.float32),
            pltpu.VMEM((G0,), jnp.int32),
            pltpu.VMEM((G1,), jnp.int32),
            pltpu.VMEM((G0,), jnp.int32),
            pltpu.VMEM((G1,), jnp.int32),
            pltpu.VMEM((G0 + 16,), jnp.int32),
            pltpu.VMEM((G1 + 16,), jnp.int32),
            pltpu.VMEM((G0 + 16,), jnp.int32),
            pltpu.VMEM((G1 + 16,), jnp.int32),
            pltpu.SemaphoreType.DMA,
            pltpu.SemaphoreType.DMA,
            pltpu.SemaphoreType.DMA,
            pltpu.SemaphoreType.DMA,
        ],
        compiler_params=pltpu.CompilerParams(
            needs_layout_passes=False,
            use_tc_tiling_on_sc=True,
        ),
    )
    def k(w_hbm, idx_hbm, out_hbm, idx_v,
          big_a0, big_a1, big_b0, big_b1, out_a, out_b,
          ids_a0, ids_a1, ids_b0, ids_b1,
          off_a0, off_a1, off_b0, off_b1,
          sia, sib, soa, sob):
        wid = lax.axis_index("s") * NC + lax.axis_index("c")
        pltpu.sync_copy(idx_hbm.at[pl.ds(wid * ipw, ipw)], idx_v)
        base = wid * bpw

        def prep(j, ids0, ids1, off0, off1):
            for kk in range(13):
                lo = kk * 16 if kk < 12 else t - 16
                v = idx_v[pl.ds(j * t + lo, 16)]
                big = lax.shift_right_logical(v, 1)
                off = lax.shift_left(v & 1, 6)
                if lo + 16 <= G0:
                    ids0[pl.ds(lo, 16)] = big
                    off0[pl.ds(lo, 16)] = off
                else:
                    ids1[pl.ds(lo - G0, 16)] = big
                    off1[pl.ds(lo - G0, 16)] = off

        def gather(b0, b1, ids0, ids1, sem):
            pltpu.async_copy(w_hbm.at[ids0], b0, sem)
            pltpu.async_copy(w_hbm.at[ids1], b1, sem)

        def wait_in(b0, b1, ids0, ids1, sem):
            pltpu.make_async_copy(w_hbm.at[ids0], b0, sem).wait()
            pltpu.make_async_copy(w_hbm.at[ids1], b1, sem).wait()

        def put(b, buf, sem):
            pltpu.async_copy(buf, out_hbm.at[base + b], sem)

        def wait_out(b, buf, sem):
            pltpu.make_async_copy(buf, out_hbm.at[base + b], sem).wait()

        prep(0, ids_a0, ids_a1, off_a0, off_a1)
        gather(big_a0, big_a1, ids_a0, ids_a1, sia)

        @pl.loop(0, bpw // 2)
        def _it(g):
            j0 = 2 * g

            prep(j0 + 1, ids_b0, ids_b1, off_b0, off_b1)
            gather(big_b0, big_b1, ids_b0, ids_b1, sib)

            wait_in(big_a0, big_a1, ids_a0, ids_a1, sia)

            @pl.when(g > 0)
            def _():
                wait_out(j0 - 2, out_a, soa)

            _normalize(big_a0, off_a0, out_a, 0, G0)
            _normalize(big_a1, off_a1, out_a, G0, G1)
            put(j0, out_a, soa)

            @pl.when(j0 + 2 < bpw)
            def _():
                prep(j0 + 2, ids_a0, ids_a1, off_a0, off_a1)
                gather(big_a0, big_a1, ids_a0, ids_a1, sia)

            wait_in(big_b0, big_b1, ids_b0, ids_b1, sib)

            @pl.when(g > 0)
            def _():
                wait_out(j0 - 1, out_b, sob)

            _normalize(big_b0, off_b0, out_b, 0, G0)
            _normalize(big_b1, off_b1, out_b, G0, G1)
            put(j0 + 1, out_b, sob)

        wait_out(bpw - 2, out_a, soa)
        wait_out(bpw - 1, out_b, sob)

    return k(w2, idx_flat)


def kernel(x, weight):
    nb, t = x.shape
    idx_flat = x.reshape(-1).astype(jnp.int32)
    w2 = weight.reshape(weight.shape[0] // 2, 2 * EMBED)
    return _sc_embed(idx_flat, w2, nb, t)


# speedup vs baseline: 1.1435x; 1.1435x over previous
import functools

import jax
import jax.numpy as jnp
from jax import lax
from jax.experimental import pallas as pl
from jax.experimental.pallas import tpu as pltpu
from jax.experimental.pallas import tpu_sc as plsc

EMBED = 64
SCALE = 8.0
NC, NS = 2, 16
CHUNK = 128


def _rsqrt_newton(x):
    i = plsc.bitcast(x, jnp.int32)
    i = jnp.int32(0x5F3759DF) - lax.shift_right_logical(i, 1)
    y = plsc.bitcast(i, jnp.float32)
    for _ in range(3):
        y = y * (jnp.float32(1.5) - jnp.float32(0.5) * x * y * y)
    return y


def _normalize(big_v, offs_v, out_v):
    lanes = lax.iota(jnp.int32, 16)
    perm = [lanes ^ (1 << k) for k in range(4)]

    @plsc.parallel_loop(0, CHUNK, unroll=8)
    def _row(r):
        o = offs_v[pl.ds(r, 16)][0]
        vs = [big_v[r, pl.ds(o + k * 16, 16)] for k in range(EMBED // 16)]
        ssv = vs[0] * vs[0]
        for v in vs[1:]:
            ssv = ssv + v * v
        for p in perm:
            ssv = ssv + jnp.take(ssv, p)
        x = jnp.maximum(ssv, jnp.float32(1e-30))
        norm = x * _rsqrt_newton(x)
        factor = jnp.float32(SCALE) / jnp.maximum(norm, jnp.float32(1e-12))
        for k, v in enumerate(vs):
            out_v[r, pl.ds(k * 16, 16)] = v * factor


def _sc_embed(idx2d, w2):
    R, C = idx2d.shape
    NW = NC * NS
    rpw = R // NW
    B = R * C

    mesh = plsc.VectorSubcoreMesh(core_axis_name="c", subcore_axis_name="s")

    @functools.partial(
        pl.kernel,
        out_type=jax.ShapeDtypeStruct((B, EMBED), jnp.float32),
        mesh=mesh,
        scratch_types=[
            pltpu.VMEM((rpw, C), jnp.int32),
            pltpu.VMEM((C, 2 * EMBED), jnp.float32),
            pltpu.VMEM((C, 2 * EMBED), jnp.float32),
            pltpu.VMEM((C, EMBED), jnp.float32),
            pltpu.VMEM((C, EMBED), jnp.float32),
            pltpu.VMEM((C,), jnp.int32),
            pltpu.VMEM((C,), jnp.int32),
            pltpu.VMEM((C + 16,), jnp.int32),
            pltpu.VMEM((C + 16,), jnp.int32),
            pltpu.SemaphoreType.DMA,
            pltpu.SemaphoreType.DMA,
            pltpu.SemaphoreType.DMA,
            pltpu.SemaphoreType.DMA,
        ],
        compiler_params=pltpu.CompilerParams(
            needs_layout_passes=False,
            use_tc_tiling_on_sc=True,
        ),
    )
    def k(w_hbm, idx_hbm, out_hbm, idx_v, big_a, big_b, out_a, out_b,
          ids_a, ids_b, off_a, off_b, sia, sib, soa, sob):
        wid = lax.axis_index("s") * NC + lax.axis_index("c")
        pltpu.sync_copy(idx_hbm.at[pl.ds(wid * rpw, rpw)], idx_v)
        base = wid * rpw

        def prep(j, ids, off):
            for kk in range(C // 16):
                v = idx_v[j, pl.ds(kk * 16, 16)]
                ids[pl.ds(kk * 16, 16)] = lax.shift_right_logical(v, 1)
                off[pl.ds(kk * 16, 16)] = lax.shift_left(v & 1, 6)

        def gather(buf, ids, sem):
            pltpu.async_copy(w_hbm.at[ids], buf, sem)

        def wait_in(buf, ids, sem):
            pltpu.make_async_copy(w_hbm.at[ids], buf, sem).wait()

        def put(j, buf, sem):
            pltpu.async_copy(buf, out_hbm.at[pl.ds((base + j) * C, C)], sem)

        def wait_out(j, buf, sem):
            pltpu.make_async_copy(
                buf, out_hbm.at[pl.ds((base + j) * C, C)], sem).wait()

        prep(0, ids_a, off_a)
        gather(big_a, ids_a, sia)

        @pl.loop(0, rpw // 2)
        def _it(g):
            j0 = 2 * g

            prep(j0 + 1, ids_b, off_b)
            gather(big_b, ids_b, sib)

            wait_in(big_a, ids_a, sia)

            @pl.when(g > 0)
            def _():
                wait_out(j0 - 2, out_a, soa)

            _normalize(big_a, off_a, out_a)
            put(j0, out_a, soa)

            @pl.when(j0 + 2 < rpw)
            def _():
                prep(j0 + 2, ids_a, off_a)
                gather(big_a, ids_a, sia)

            wait_in(big_b, ids_b, sib)

            @pl.when(g > 0)
            def _():
                wait_out(j0 - 1, out_b, sob)

            _normalize(big_b, off_b, out_b)
            put(j0 + 1, out_b, sob)

        wait_out(rpw - 2, out_a, soa)
        wait_out(rpw - 1, out_b, sob)

    return k(w2, idx2d)


def kernel(x, weight):
    nb, t = x.shape
    b = nb * t
    idx2d = x.reshape(b // CHUNK, CHUNK).astype(jnp.int32)
    w2 = weight.reshape(weight.shape[0] // 2, 2 * EMBED)
    out = _sc_embed(idx2d, w2)
    return out.reshape(nb, t, EMBED)
